# baseline (speedup 1.0000x reference)
import functools

import jax
import jax.numpy as jnp
import numpy as np
from jax import lax
from jax.experimental import pallas as pl
from jax.experimental.pallas import tpu as pltpu
from jax.experimental.pallas import tpu_sc as plsc

_NT = 1000
_B = 64
_C = 3
_N = 2048
_D = _C * _N
_NW = 32
_BPW = _B // _NW
_L = 16
_CH = _D // _L


def _make_coef_table() -> np.ndarray:
    betas = np.linspace(0.0001, 0.02, _NT).astype(np.float64)
    alphas = 1.0 - betas
    ac = np.cumprod(alphas, axis=0)
    acp = np.append(1.0, ac[:-1])
    sra = np.sqrt(1.0 / ac)
    srm1 = np.sqrt(1.0 / ac - 1.0)
    pv = betas * (1.0 - acp) / (1.0 - ac)
    c1 = betas * np.sqrt(acp) / (1.0 - ac)
    c2 = (1.0 - acp) * np.sqrt(alphas) / (1.0 - ac)
    plvc32 = np.log(np.maximum(pv, 1e-20)).astype(np.float32)
    sig = np.exp(0.5 * plvc32.astype(np.float64))
    sig[0] = 0.0
    tbl = np.zeros((_NT, 8), dtype=np.float32)
    tbl[:, 0] = sra
    tbl[:, 1] = srm1
    tbl[:, 2] = c1
    tbl[:, 3] = c2
    tbl[:, 4] = sig
    return tbl.reshape(-1)


_COEF = _make_coef_table()


@functools.lru_cache(maxsize=None)
def _build_p_sample_sc():
    mesh = plsc.VectorSubcoreMesh(core_axis_name="c", subcore_axis_name="s")
    return pl.kernel(
        functools.partial(_p_sample_sc_body, mesh.num_cores),
        out_type=(
            jax.ShapeDtypeStruct((_B, _D), jnp.float32),
            jax.ShapeDtypeStruct((_B, _D), jnp.float32),
        ),
        mesh=mesh,
        compiler_params=pltpu.CompilerParams(needs_layout_passes=False),
        scratch_types=[
            pltpu.VMEM((_B,), jnp.int32),
            pltpu.VMEM((_NT * 8,), jnp.float32),
            pltpu.VMEM((_BPW, _D), jnp.float32),
            pltpu.VMEM((_BPW, _D), jnp.float32),
            pltpu.VMEM((_BPW, _D), jnp.float32),
        ],
    )


def _p_sample_sc_body(num_cores, data_hbm, t_hbm, mo_hbm, noise_hbm, coef_hbm,
                      sample_hbm, xrec_hbm,
                      t_v, coef_v, d_v, m_v, n_v):
    wid = lax.axis_index("s") * num_cores + lax.axis_index("c")
    base = wid * _BPW
    pltpu.sync_copy(t_hbm, t_v)
    pltpu.sync_copy(coef_hbm, coef_v)
    pltpu.sync_copy(data_hbm.at[pl.ds(base, _BPW)], d_v)
    pltpu.sync_copy(mo_hbm.at[pl.ds(base, _BPW)], m_v)
    pltpu.sync_copy(noise_hbm.at[pl.ds(base, _BPW)], n_v)
    for r in range(_BPW):
        rowv = jnp.full((_L,), base + r, jnp.int32)
        tv = plsc.load_gather(t_v, [rowv])

        tv8 = tv * 8

        def col(k):
            return plsc.load_gather(coef_v, [tv8 + k])

        sra, srm1, c1, c2, sg = col(0), col(1), col(2), col(3), col(4)

        def body(j, carry):
            sl = pl.ds(j * _L, _L)
            d = d_v[r, sl]
            m = m_v[r, sl]
            z = n_v[r, sl]
            xr = sra * d - srm1 * m
            xr = jnp.minimum(jnp.maximum(xr, -0.5), 0.5)
            m_v[r, sl] = xr
            n_v[r, sl] = c1 * xr + c2 * d + sg * z
            return carry

        lax.fori_loop(0, _CH, body, 0)
    pltpu.sync_copy(n_v, sample_hbm.at[pl.ds(base, _BPW)])
    pltpu.sync_copy(m_v, xrec_hbm.at[pl.ds(base, _BPW)])


def kernel(data, t, model_output, noise):
    d2 = data.reshape(_B, _D)
    m2 = model_output.reshape(_B, _D)
    n2 = noise.reshape(_B, _D)
    sample, xrec = _build_p_sample_sc()(d2, t.astype(jnp.int32), m2, n2,
                                        jnp.asarray(_COEF))
    return sample.reshape(_B, _C, _N), xrec.reshape(_B, _C, _N)


# speedup vs baseline: 1.1850x; 1.1850x over previous
import functools

import jax
import jax.numpy as jnp
import numpy as np
from jax import lax
from jax.experimental import pallas as pl
from jax.experimental.pallas import tpu as pltpu
from jax.experimental.pallas import tpu_sc as plsc

_NT = 1000
_B = 64
_C = 3
_N = 2048
_D = _C * _N
_NW = 32
_BPW = _B // _NW
_L = 16
_CH = _D // _L


def _make_coef_table() -> np.ndarray:
    betas = np.linspace(0.0001, 0.02, _NT).astype(np.float64)
    alphas = 1.0 - betas
    ac = np.cumprod(alphas, axis=0)
    acp = np.append(1.0, ac[:-1])
    sra = np.sqrt(1.0 / ac)
    srm1 = np.sqrt(1.0 / ac - 1.0)
    pv = betas * (1.0 - acp) / (1.0 - ac)
    c1 = betas * np.sqrt(acp) / (1.0 - ac)
    c2 = (1.0 - acp) * np.sqrt(alphas) / (1.0 - ac)
    plvc32 = np.log(np.maximum(pv, 1e-20)).astype(np.float32)
    sig = np.exp(0.5 * plvc32.astype(np.float64))
    sig[0] = 0.0
    tbl = np.zeros((_NT, 8), dtype=np.float32)
    tbl[:, 0] = sra
    tbl[:, 1] = srm1
    tbl[:, 2] = c1
    tbl[:, 3] = c2
    tbl[:, 4] = sig
    return tbl.reshape(-1)


_COEF = _make_coef_table()


@functools.lru_cache(maxsize=None)
def _build_p_sample_sc():
    mesh = plsc.VectorSubcoreMesh(core_axis_name="c", subcore_axis_name="s")
    return pl.kernel(
        functools.partial(_p_sample_sc_body, mesh.num_cores),
        out_type=(
            jax.ShapeDtypeStruct((_B, _D), jnp.float32),
            jax.ShapeDtypeStruct((_B, _D), jnp.float32),
        ),
        mesh=mesh,
        compiler_params=pltpu.CompilerParams(needs_layout_passes=False),
        scratch_types=[
            pltpu.VMEM((_B,), jnp.int32),
            pltpu.VMEM((_NT * 8,), jnp.float32),
            pltpu.VMEM((_BPW, _D), jnp.float32),
            pltpu.VMEM((_BPW, _D), jnp.float32),
            pltpu.VMEM((_BPW, _D), jnp.float32),
            pltpu.SemaphoreType.DMA,
            pltpu.SemaphoreType.DMA,
            pltpu.SemaphoreType.DMA,
            pltpu.SemaphoreType.DMA,
            pltpu.SemaphoreType.DMA,
        ],
    )


def _p_sample_sc_body(num_cores, data_hbm, t_hbm, mo_hbm, noise_hbm, coef_hbm,
                      sample_hbm, xrec_hbm,
                      t_v, coef_v, d_v, m_v, n_v,
                      sem_t, sem_c, sem_d, sem_m, sem_n):
    wid = lax.axis_index("s") * num_cores + lax.axis_index("c")
    base = wid * _BPW
    sl_b = pl.ds(base, _BPW)
    cp_t = pltpu.async_copy(t_hbm, t_v, sem_t)
    cp_c = pltpu.async_copy(coef_hbm, coef_v, sem_c)
    cp_d = pltpu.async_copy(data_hbm.at[sl_b], d_v, sem_d)
    cp_m = pltpu.async_copy(mo_hbm.at[sl_b], m_v, sem_m)
    cp_n = pltpu.async_copy(noise_hbm.at[sl_b], n_v, sem_n)
    cp_t.wait()
    cp_c.wait()
    coefs = []
    for r in range(_BPW):
        rowv = jnp.full((_L,), base + r, jnp.int32)
        tv8 = plsc.load_gather(t_v, [rowv]) * 8
        coefs.append([plsc.load_gather(coef_v, [tv8 + k]) for k in range(5)])
    cp_d.wait()
    cp_m.wait()
    cp_n.wait()
    for r in range(_BPW):
        sra, srm1, c1, c2, sg = coefs[r]

        @plsc.parallel_loop(0, _CH, step=1, unroll=8)
        def _(j):
            sl = pl.ds(j * _L, _L)
            d = d_v[r, sl]
            m = m_v[r, sl]
            z = n_v[r, sl]
            xr = sra * d - srm1 * m
            xr = jnp.minimum(jnp.maximum(xr, -0.5), 0.5)
            m_v[r, sl] = xr
            n_v[r, sl] = c1 * xr + c2 * d + sg * z

    cp_s = pltpu.async_copy(n_v, sample_hbm.at[sl_b], sem_n)
    cp_x = pltpu.async_copy(m_v, xrec_hbm.at[sl_b], sem_m)
    cp_s.wait()
    cp_x.wait()


def kernel(data, t, model_output, noise):
    d2 = data.reshape(_B, _D)
    m2 = model_output.reshape(_B, _D)
    n2 = noise.reshape(_B, _D)
    sample, xrec = _build_p_sample_sc()(d2, t.astype(jnp.int32), m2, n2,
                                        jnp.asarray(_COEF))
    return sample.reshape(_B, _C, _N), xrec.reshape(_B, _C, _N)


# speedup vs baseline: 1.1881x; 1.0026x over previous
import functools

import jax
import jax.numpy as jnp
import numpy as np
from jax import lax
from jax.experimental import pallas as pl
from jax.experimental.pallas import tpu as pltpu
from jax.experimental.pallas import tpu_sc as plsc

_NT = 1000
_B = 64
_C = 3
_N = 2048
_D = _C * _N
_NW = 32
_BPW = _B // _NW
_L = 16
_CH = _D // _L


def _make_coef_table() -> np.ndarray:
    betas = np.linspace(0.0001, 0.02, _NT).astype(np.float64)
    alphas = 1.0 - betas
    ac = np.cumprod(alphas, axis=0)
    acp = np.append(1.0, ac[:-1])
    sra = np.sqrt(1.0 / ac)
    srm1 = np.sqrt(1.0 / ac - 1.0)
    pv = betas * (1.0 - acp) / (1.0 - ac)
    c1 = betas * np.sqrt(acp) / (1.0 - ac)
    c2 = (1.0 - acp) * np.sqrt(alphas) / (1.0 - ac)
    plvc32 = np.log(np.maximum(pv, 1e-20)).astype(np.float32)
    sig = np.exp(0.5 * plvc32.astype(np.float64))
    sig[0] = 0.0
    tbl = np.zeros((_NT, 8), dtype=np.float32)
    tbl[:, 0] = sra
    tbl[:, 1] = srm1
    tbl[:, 2] = c1
    tbl[:, 3] = c2
    tbl[:, 4] = sig
    return tbl.reshape(-1)


_COEF = _make_coef_table()


@functools.lru_cache(maxsize=None)
def _build_p_sample_sc():
    mesh = plsc.VectorSubcoreMesh(core_axis_name="c", subcore_axis_name="s")
    return pl.kernel(
        functools.partial(_p_sample_sc_body, mesh.num_cores),
        out_type=(
            jax.ShapeDtypeStruct((_B, _D), jnp.float32),
            jax.ShapeDtypeStruct((_B, _D), jnp.float32),
        ),
        mesh=mesh,
        compiler_params=pltpu.CompilerParams(needs_layout_passes=False,
                                             skip_device_barrier=True),
        scratch_types=[
            pltpu.VMEM((_B,), jnp.int32),
            pltpu.VMEM((_NT * 8,), jnp.float32),
            pltpu.VMEM((_BPW, _D), jnp.float32),
            pltpu.VMEM((_BPW, _D), jnp.float32),
            pltpu.VMEM((_BPW, _D), jnp.float32),
            pltpu.SemaphoreType.DMA,
            pltpu.SemaphoreType.DMA,
            pltpu.SemaphoreType.DMA,
            pltpu.SemaphoreType.DMA,
            pltpu.SemaphoreType.DMA,
        ],
    )


def _p_sample_sc_body(num_cores, data_hbm, t_hbm, mo_hbm, noise_hbm, coef_hbm,
                      sample_hbm, xrec_hbm,
                      t_v, coef_v, d_v, m_v, n_v,
                      sem_t, sem_c, sem_d, sem_m, sem_n):
    wid = lax.axis_index("s") * num_cores + lax.axis_index("c")
    base = wid * _BPW
    sl_b = pl.ds(base, _BPW)
    cp_t = pltpu.async_copy(t_hbm, t_v, sem_t)
    cp_c = pltpu.async_copy(coef_hbm, coef_v, sem_c)
    cp_d = pltpu.async_copy(data_hbm.at[sl_b], d_v, sem_d)
    cp_m = pltpu.async_copy(mo_hbm.at[sl_b], m_v, sem_m)
    cp_n = pltpu.async_copy(noise_hbm.at[sl_b], n_v, sem_n)
    cp_t.wait()
    cp_c.wait()
    coefs = []
    for r in range(_BPW):
        rowv = jnp.full((_L,), base + r, jnp.int32)
        tv8 = plsc.load_gather(t_v, [rowv]) * 8
        coefs.append([plsc.load_gather(coef_v, [tv8 + k]) for k in range(5)])
    cp_d.wait()
    cp_m.wait()
    cp_n.wait()
    for r in range(_BPW):
        sra, srm1, c1, c2, sg = coefs[r]

        @plsc.parallel_loop(0, _CH, step=1, unroll=8)
        def _(j):
            sl = pl.ds(j * _L, _L)
            d = d_v[r, sl]
            m = m_v[r, sl]
            z = n_v[r, sl]
            xr = sra * d - srm1 * m
            xr = jnp.minimum(jnp.maximum(xr, -0.5), 0.5)
            m_v[r, sl] = xr
            n_v[r, sl] = c1 * xr + c2 * d + sg * z

    cp_s = pltpu.async_copy(n_v, sample_hbm.at[sl_b], sem_n)
    cp_x = pltpu.async_copy(m_v, xrec_hbm.at[sl_b], sem_m)
    cp_s.wait()
    cp_x.wait()


def kernel(data, t, model_output, noise):
    d2 = data.reshape(_B, _D)
    m2 = model_output.reshape(_B, _D)
    n2 = noise.reshape(_B, _D)
    sample, xrec = _build_p_sample_sc()(d2, t.astype(jnp.int32), m2, n2,
                                        jnp.asarray(_COEF))
    return sample.reshape(_B, _C, _N), xrec.reshape(_B, _C, _N)


# speedup vs baseline: 1.1975x; 1.0079x over previous
import functools

import jax
import jax.numpy as jnp
import numpy as np
from jax import lax
from jax.experimental import pallas as pl
from jax.experimental.pallas import tpu as pltpu
from jax.experimental.pallas import tpu_sc as plsc

_NT = 1000
_B = 64
_C = 3
_N = 2048
_D = _C * _N
_NW = 32
_BPW = _B // _NW
_L = 16
_CH = _D // _L


def _make_coef_table() -> np.ndarray:
    betas = np.linspace(0.0001, 0.02, _NT).astype(np.float64)
    alphas = 1.0 - betas
    ac = np.cumprod(alphas, axis=0)
    acp = np.append(1.0, ac[:-1])
    sra = np.sqrt(1.0 / ac)
    srm1 = np.sqrt(1.0 / ac - 1.0)
    pv = betas * (1.0 - acp) / (1.0 - ac)
    c1 = betas * np.sqrt(acp) / (1.0 - ac)
    c2 = (1.0 - acp) * np.sqrt(alphas) / (1.0 - ac)
    plvc32 = np.log(np.maximum(pv, 1e-20)).astype(np.float32)
    sig = np.exp(0.5 * plvc32.astype(np.float64))
    sig[0] = 0.0
    tbl = np.zeros((_NT, 8), dtype=np.float32)
    tbl[:, 0] = sra
    tbl[:, 1] = srm1
    tbl[:, 2] = c1
    tbl[:, 3] = c2
    tbl[:, 4] = sig
    return tbl.reshape(-1)


_COEF = _make_coef_table()


@functools.lru_cache(maxsize=None)
def _build_p_sample_sc():
    mesh = plsc.VectorSubcoreMesh(core_axis_name="c", subcore_axis_name="s")
    return pl.kernel(
        functools.partial(_p_sample_sc_body, mesh.num_cores),
        out_type=(
            jax.ShapeDtypeStruct((_B, _D), jnp.float32),
            jax.ShapeDtypeStruct((_B, _D), jnp.float32),
        ),
        mesh=mesh,
        compiler_params=pltpu.CompilerParams(needs_layout_passes=False,
                                             skip_device_barrier=True),
        scratch_types=[
            pltpu.VMEM((_B,), jnp.int32),
            pltpu.VMEM((_NT * 8,), jnp.float32),
            pltpu.VMEM((_BPW, _D), jnp.float32),
            pltpu.VMEM((_BPW, _D), jnp.float32),
            pltpu.VMEM((_BPW, _D), jnp.float32),
            pltpu.SemaphoreType.DMA,
            pltpu.SemaphoreType.DMA,
            pltpu.SemaphoreType.DMA,
            pltpu.SemaphoreType.DMA,
            pltpu.SemaphoreType.DMA,
        ],
    )


def _p_sample_sc_body(num_cores, data_hbm, t_hbm, mo_hbm, noise_hbm, coef_hbm,
                      sample_hbm, xrec_hbm,
                      t_v, coef_v, d_v, m_v, n_v,
                      sem_t, sem_c, sem_d, sem_m, sem_n):
    wid = lax.axis_index("s") * num_cores + lax.axis_index("c")
    base = wid * _BPW
    sl_b = pl.ds(base, _BPW)
    cp_t = pltpu.async_copy(t_hbm, t_v, sem_t)
    cp_c = pltpu.async_copy(coef_hbm, coef_v, sem_c)
    cp_d = pltpu.async_copy(data_hbm.at[sl_b], d_v, sem_d)
    cp_m = pltpu.async_copy(mo_hbm.at[sl_b], m_v, sem_m)
    cp_n = pltpu.async_copy(noise_hbm.at[sl_b], n_v, sem_n)
    cp_t.wait()
    cp_c.wait()
    coefs = []
    for r in range(_BPW):
        rowv = jnp.full((_L,), base + r, jnp.int32)
        tv8 = plsc.load_gather(t_v, [rowv]) * 8
        coefs.append([plsc.load_gather(coef_v, [tv8 + k]) for k in range(5)])
    cp_d.wait()
    cp_m.wait()
    cp_n.wait()
    for r in range(_BPW):
        sra, srm1, c1, c2, sg = coefs[r]

        @plsc.parallel_loop(0, _CH, step=1, unroll=2)
        def _(j):
            sl = pl.ds(j * _L, _L)
            d = d_v[r, sl]
            m = m_v[r, sl]
            z = n_v[r, sl]
            xr = sra * d - srm1 * m
            xr = jnp.minimum(jnp.maximum(xr, -0.5), 0.5)
            m_v[r, sl] = xr
            n_v[r, sl] = c1 * xr + c2 * d + sg * z

    cp_s = pltpu.async_copy(n_v, sample_hbm.at[sl_b], sem_n)
    cp_x = pltpu.async_copy(m_v, xrec_hbm.at[sl_b], sem_m)
    cp_s.wait()
    cp_x.wait()


def kernel(data, t, model_output, noise):
    d2 = data.reshape(_B, _D)
    m2 = model_output.reshape(_B, _D)
    n2 = noise.reshape(_B, _D)
    sample, xrec = _build_p_sample_sc()(d2, t.astype(jnp.int32), m2, n2,
                                        jnp.asarray(_COEF))
    return sample.reshape(_B, _C, _N), xrec.reshape(_B, _C, _N)


# speedup vs baseline: 1.5957x; 1.3326x over previous
import functools

import jax
import jax.numpy as jnp
import numpy as np
from jax import lax
from jax.experimental import pallas as pl

_NT = 1000
_B = 64
_C = 3
_N = 2048
_BBLK = 8
_GRID = _B // _BBLK


def _make_coef_table() -> np.ndarray:
    betas = np.linspace(0.0001, 0.02, _NT).astype(np.float64)
    alphas = 1.0 - betas
    ac = np.cumprod(alphas, axis=0)
    acp = np.append(1.0, ac[:-1])
    sra = np.sqrt(1.0 / ac)
    srm1 = np.sqrt(1.0 / ac - 1.0)
    pv = betas * (1.0 - acp) / (1.0 - ac)
    c1 = betas * np.sqrt(acp) / (1.0 - ac)
    c2 = (1.0 - acp) * np.sqrt(alphas) / (1.0 - ac)
    plvc32 = np.log(np.maximum(pv, 1e-20)).astype(np.float32)
    sig = np.exp(0.5 * plvc32.astype(np.float64))
    sig[0] = 0.0
    tbl = np.zeros((_NT, 8), dtype=np.float32)
    tbl[:, 0] = sra
    tbl[:, 1] = srm1
    tbl[:, 2] = c1
    tbl[:, 3] = c2
    tbl[:, 4] = sig
    return tbl


_COEF = _make_coef_table()


def _p_sample_body(t_ref, coef_ref, d_ref, m_ref, z_ref, s_ref, x_ref):
    tb = t_ref[...]
    oh = (tb == lax.broadcasted_iota(jnp.int32, (_BBLK, _NT), 1)
          ).astype(jnp.float32)
    cf = jnp.dot(oh, coef_ref[...], preferred_element_type=jnp.float32)
    d = d_ref[...]
    m = m_ref[...]
    z = z_ref[...]
    sra = cf[:, 0][:, None, None]
    srm1 = cf[:, 1][:, None, None]
    c1 = cf[:, 2][:, None, None]
    c2 = cf[:, 3][:, None, None]
    sg = cf[:, 4][:, None, None]
    xr = sra * d - srm1 * m
    xr = jnp.minimum(jnp.maximum(xr, -0.5), 0.5)
    x_ref[...] = xr
    s_ref[...] = c1 * xr + c2 * d + sg * z


@jax.jit
def _p_sample(data, t2, model_output, noise, coef):
    blk = pl.BlockSpec((_BBLK, _C, _N), lambda i: (i, 0, 0))
    return pl.pallas_call(
        _p_sample_body,
        grid=(_GRID,),
        in_specs=[
            pl.BlockSpec((_BBLK, 1), lambda i: (i, 0)),
            pl.BlockSpec((_NT, 8), lambda i: (0, 0)),
            blk,
            blk,
            blk,
        ],
        out_specs=[blk, blk],
        out_shape=[
            jax.ShapeDtypeStruct((_B, _C, _N), jnp.float32),
            jax.ShapeDtypeStruct((_B, _C, _N), jnp.float32),
        ],
    )(t2, coef, data, model_output, noise)


def kernel(data, t, model_output, noise):
    t2 = t.astype(jnp.int32)[:, None]
    sample, xrec = _p_sample(data, t2, model_output, noise,
                             jnp.asarray(_COEF))
    return sample, xrec
